# baseline (speedup 1.0000x reference)
import functools

import jax
import jax.numpy as jnp
from jax import lax
from jax.experimental import pallas as pl
from jax.experimental.pallas import tpu as pltpu
from jax.experimental.pallas import tpu_sc as plsc

VOCAB = 100000
D = 128
SEQ = 4096
W = 50

_info = plsc.get_sparse_core_info()
_NC, _NS = _info.num_cores, _info.num_subcores
_NW = _NC * _NS
_B = SEQ * W
_BPW = _B // _NW
_C = 128
_NCHUNK = _BPW // _C
_NBUF = 6


def _make_kernel():
    mesh = plsc.VectorSubcoreMesh(core_axis_name="c", subcore_axis_name="s")

    @functools.partial(
        pl.kernel,
        out_type=jax.ShapeDtypeStruct((_B, D), jnp.float32),
        mesh=mesh,
        scratch_types=[
            pltpu.VMEM((_NCHUNK, _C), jnp.int32),
            pltpu.VMEM((_NBUF, _C, D), jnp.float32),
            pltpu.SemaphoreType.DMA,
            pltpu.SemaphoreType.DMA,
        ],
        compiler_params=pltpu.CompilerParams(
            use_tc_tiling_on_sc=True,
            disable_bounds_checks=True,
            skip_device_barrier=True,
        ),
    )
    def emb(idx_hbm, table_hbm, out_hbm, idx_v, rows_v, gsem, wsem):
        wid = lax.axis_index("s") * _NC + lax.axis_index("c")
        col = wid * _C
        pltpu.sync_copy(idx_hbm.at[pl.ds(0, W), pl.ds(col, _C)], idx_v)

        for j in range(4):
            pltpu.async_copy(table_hbm.at[idx_v.at[j]], rows_v.at[j], gsem)

        def body(i, carry):
            b = lax.rem(i, _NBUF)

            @pl.when(i >= 2)
            def _():
                pltpu.make_async_copy(
                    out_hbm.at[pl.ds(0, _C)], rows_v.at[b], wsem
                ).wait()

            @pl.when(i + 4 < _NCHUNK)
            def _():
                pltpu.async_copy(
                    table_hbm.at[idx_v.at[i + 4]],
                    rows_v.at[lax.rem(i + 4, _NBUF)],
                    gsem,
                )

            pltpu.make_async_copy(
                out_hbm.at[pl.ds(0, _C)], rows_v.at[b], gsem
            ).wait()
            pltpu.async_copy(
                rows_v.at[b], out_hbm.at[pl.ds(i * SEQ + col, _C)], wsem
            )
            return carry

        lax.fori_loop(0, _NCHUNK, body, 0)
        for j in range(2):
            pltpu.make_async_copy(
                out_hbm.at[pl.ds(0, _C)], rows_v.at[j], wsem
            ).wait()

    return emb


_emb = _make_kernel()


def kernel(x, table):
    idx = x.T.astype(jnp.int32)
    out = _emb(idx, table)
    return out.reshape(W, SEQ, D).transpose(1, 0, 2)


# speedup vs baseline: 1.0054x; 1.0054x over previous
import functools

import jax
import jax.numpy as jnp
from jax import lax
from jax.experimental import pallas as pl
from jax.experimental.pallas import tpu as pltpu
from jax.experimental.pallas import tpu_sc as plsc

VOCAB = 100000
D = 128
SEQ = 4096
W = 50

_info = plsc.get_sparse_core_info()
_NC, _NS = _info.num_cores, _info.num_subcores
_NW = _NC * _NS
_B = SEQ * W
_BPW = _B // _NW
_C = 128
_NCHUNK = _BPW // _C
_NBUF = 5


def _make_kernel():
    mesh = plsc.VectorSubcoreMesh(core_axis_name="c", subcore_axis_name="s")

    @functools.partial(
        pl.kernel,
        out_type=jax.ShapeDtypeStruct((_B, D), jnp.float32),
        mesh=mesh,
        scratch_types=[
            pltpu.VMEM((_NCHUNK, _C), jnp.int32),
            pltpu.VMEM((_NBUF, _C, D), jnp.float32),
            pltpu.SemaphoreType.DMA,
            pltpu.SemaphoreType.DMA,
        ],
        compiler_params=pltpu.CompilerParams(
            use_tc_tiling_on_sc=True,
            disable_bounds_checks=True,
            skip_device_barrier=True,
        ),
    )
    def emb(idx_hbm, table_hbm, out_hbm, idx_v, rows_v, gsem, wsem):
        wid = lax.axis_index("s") * _NC + lax.axis_index("c")
        col = wid * _C
        pltpu.sync_copy(idx_hbm.at[pl.ds(0, W), pl.ds(col, _C)], idx_v)

        for j in range(3):
            pltpu.async_copy(table_hbm.at[idx_v.at[j]], rows_v.at[j], gsem)

        def body(i, carry):
            b = lax.rem(i, _NBUF)

            @pl.when(i >= 2)
            def _():
                pltpu.make_async_copy(
                    out_hbm.at[pl.ds(0, _C)], rows_v.at[b], wsem
                ).wait()

            @pl.when(i + 3 < _NCHUNK)
            def _():
                pltpu.async_copy(
                    table_hbm.at[idx_v.at[i + 3]],
                    rows_v.at[lax.rem(i + 3, _NBUF)],
                    gsem,
                )

            pltpu.make_async_copy(
                out_hbm.at[pl.ds(0, _C)], rows_v.at[b], gsem
            ).wait()
            pltpu.async_copy(
                rows_v.at[b], out_hbm.at[pl.ds(i * SEQ + col, _C)], wsem
            )
            return carry

        lax.fori_loop(0, _NCHUNK, body, 0)
        for j in range(2):
            pltpu.make_async_copy(
                out_hbm.at[pl.ds(0, _C)], rows_v.at[j], wsem
            ).wait()

    return emb


_emb = _make_kernel()


def kernel(x, table):
    idx = x.T.astype(jnp.int32)
    out = _emb(idx, table)
    return out.reshape(W, SEQ, D).transpose(1, 0, 2)
